# baseline (speedup 1.0000x reference)
import functools

import jax
import jax.numpy as jnp
from jax import lax
from jax.experimental import pallas as pl
from jax.experimental.pallas import tpu as pltpu
from jax.experimental.pallas import tpu_sc as plsc

N = 10000
E = 320000
D = 128
NC, NS = 2, 16
NW = NC * NS
NP = 10240
B = 64
EPT = 10240
CH = EPT // B
BG = 128
CG = EPT // BG
EPAD = EPT * NW
RPT = NP // NS
TOTCH = EPAD // B
CH0 = 176
CH1 = (TOTCH - NS * CH0) // NS
CHMAX = max(CH0, CH1)
GMAX = CHMAX // 2
TOTG = TOTCH // 2 + 32
TOTB = TOTCH + 64
EPAD2 = TOTG * BG
NPA = 10112
RPA = NPA // NS

_mesh = plsc.VectorSubcoreMesh(
    core_axis_name="c", subcore_axis_name="s", num_cores=NC, num_subcores=NS)


@functools.partial(
    pl.kernel,
    out_type=jax.ShapeDtypeStruct((NC, NP), jnp.float32),
    mesh=_mesh,
    scratch_types=[
        pltpu.VMEM((CH, B), jnp.int32),
        pltpu.VMEM((B,), jnp.float32),
        pltpu.VMEM_SHARED((NP,), jnp.float32),
    ],
)
def _deg_kernel(dst_hbm, zeros1_hbm, ones_hbm, deg_out, dst_v, ones_v, deg_sp):
    c = lax.axis_index("c")
    s = lax.axis_index("s")
    wid = c * NS + s
    pltpu.sync_copy(dst_hbm.at[wid], dst_v)
    pltpu.sync_copy(ones_hbm, ones_v)
    pltpu.sync_copy(zeros1_hbm.at[pl.ds(s * RPT, RPT)],
                    deg_sp.at[pl.ds(s * RPT, RPT)])
    plsc.subcore_barrier()

    def body(j, carry):
        pltpu.sync_copy(ones_v, deg_sp.at[dst_v.at[j]], add=True)
        return carry

    lax.fori_loop(0, CH, body, 0)
    plsc.subcore_barrier()
    pltpu.sync_copy(deg_sp.at[pl.ds(s * RPT, RPT)],
                    deg_out.at[c, pl.ds(s * RPT, RPT)])


@functools.partial(
    pl.kernel,
    out_type=jax.ShapeDtypeStruct((NC, NP, D), jnp.float32),
    mesh=_mesh,
    scratch_types=[
        pltpu.VMEM((4, 1, B), jnp.int32),
        pltpu.VMEM((CHMAX, B), jnp.int32),
        pltpu.VMEM((2, B, D), jnp.float32),
        pltpu.VMEM_SHARED((NPA, D), jnp.float32),
        pltpu.SemaphoreType.DMA,
        pltpu.SemaphoreType.DMA,
    ],
)
def _agg_kernel(g_hbm, srcb_hbm, dstb_hbm, zeros2_hbm, agg_out,
                src_r, dst_v, rows, agg_sp, gsem, isem):
    c = lax.axis_index("c")
    s = lax.axis_index("s")
    myc = jnp.where(c == 0, CH0, CH1)
    bbase = pl.multiple_of(
        jnp.where(c == 0, s * CH0, NS * CH0 + s * CH1), 8)
    pltpu.sync_copy(dstb_hbm.at[pl.ds(bbase, CHMAX)], dst_v)
    pltpu.sync_copy(zeros2_hbm.at[pl.ds(s * RPA, RPA)],
                    agg_sp.at[pl.ds(s * RPA, RPA)])
    plsc.subcore_barrier()

    def load_idx(j):
        pltpu.async_copy(srcb_hbm.at[bbase + j], src_r.at[j % 4], isem)

    def wait_idx(j):
        pltpu.make_async_copy(srcb_hbm.at[bbase + j], src_r.at[j % 4],
                              isem).wait()

    def gather(j):
        pltpu.async_copy(g_hbm.at[src_r.at[j % 4, 0]], rows.at[j % 2], gsem)

    load_idx(0)
    load_idx(1)
    wait_idx(0)
    gather(0)

    def body(j, carry):
        @pl.when(j + 1 < myc)
        def _():
            wait_idx(j + 1)
            gather(j + 1)

        @pl.when(j + 2 < myc)
        def _():
            load_idx(j + 2)

        pltpu.make_async_copy(g_hbm.at[src_r.at[j % 4, 0]], rows.at[j % 2],
                              gsem).wait()
        pltpu.sync_copy(rows.at[j % 2], agg_sp.at[dst_v.at[j]], add=True)
        return carry

    lax.fori_loop(0, myc, body, 0)
    plsc.subcore_barrier()
    pltpu.sync_copy(agg_sp.at[pl.ds(s * RPA, RPA)],
                    agg_out.at[c, pl.ds(s * RPA, RPA)])


BM = 1024
_GRID = NP // BM


def _mm_body(x_ref, w_ref, dr_ref, g_ref, dinv_ref):
    deg = dr_ref[0] + dr_ref[1] + 1.0
    dinv = lax.rsqrt(deg)
    h = jnp.dot(x_ref[...], w_ref[...], preferred_element_type=jnp.float32)
    g_ref[...] = dinv * h
    dinv_ref[...] = dinv


_mm = pl.pallas_call(
    _mm_body,
    grid=(_GRID,),
    in_specs=[
        pl.BlockSpec((BM, D), lambda i: (i, 0)),
        pl.BlockSpec((D, D), lambda i: (0, 0)),
        pl.BlockSpec((NC, BM, 1), lambda i: (0, i, 0)),
    ],
    out_specs=[
        pl.BlockSpec((BM, D), lambda i: (i, 0)),
        pl.BlockSpec((BM, 1), lambda i: (i, 0)),
    ],
    out_shape=[
        jax.ShapeDtypeStruct((NP, D), jnp.float32),
        jax.ShapeDtypeStruct((NP, 1), jnp.float32),
    ],
)


def _fin_body(p_ref, g_ref, dinv_ref, b_ref, o_ref):
    ssum = p_ref[0] + p_ref[1] + g_ref[...]
    o_ref[...] = jnp.maximum(dinv_ref[...] * ssum + b_ref[...], 0.0)


_fin = pl.pallas_call(
    _fin_body,
    grid=(_GRID,),
    in_specs=[
        pl.BlockSpec((NC, BM, D), lambda i: (0, i, 0)),
        pl.BlockSpec((BM, D), lambda i: (i, 0)),
        pl.BlockSpec((BM, 1), lambda i: (i, 0)),
        pl.BlockSpec((1, D), lambda i: (0, 0)),
    ],
    out_specs=pl.BlockSpec((BM, D), lambda i: (i, 0)),
    out_shape=jax.ShapeDtypeStruct((NP, D), jnp.float32),
)


def kernel(x, edge_index, W, b):
    src = edge_index[0]
    dst = edge_index[1]
    pad = EPAD2 - E
    fill = jnp.full((pad,), N, jnp.int32)
    srcp = jnp.concatenate([src, fill]).reshape(TOTB, 1, B)
    dstp = jnp.concatenate([dst, fill])
    dstp_deg = dstp[:EPAD].reshape(NW, CH, B)
    dstp_agg = dstp.reshape(TOTB, B)
    xp = jnp.pad(x, ((0, NP - N), (0, 0)))
    zeros1 = jnp.zeros((NP,), jnp.float32)
    zeros2 = jnp.zeros((NP, D), jnp.float32)
    ones = jnp.ones((B,), jnp.float32)

    deg = _deg_kernel(dstp_deg, zeros1, ones)
    g, dinv = _mm(xp, W, deg.reshape(NC, NP, 1))
    agg = _agg_kernel(g, srcp, dstp_agg, zeros2)
    out = _fin(agg, g, dinv, b.reshape(1, D))
    return out[:N]


# speedup vs baseline: 1.0090x; 1.0090x over previous
import functools

import jax
import jax.numpy as jnp
from jax import lax
from jax.experimental import pallas as pl
from jax.experimental.pallas import tpu as pltpu
from jax.experimental.pallas import tpu_sc as plsc

N = 10000
E = 320000
D = 128
NC, NS = 2, 16
NW = NC * NS
NP = 10240
B = 64
EPT = 10240
CH = EPT // B
BG = 128
CG = EPT // BG
EPAD = EPT * NW
RPT = NP // NS
TOTCH = EPAD // B
CH0 = 224
CH1 = (TOTCH - NS * CH0) // NS
CHMAX = max(CH0, CH1)
GMAX = CHMAX // 2
TOTG = TOTCH // 2 + 32
TOTB = TOTCH + 64
EPAD2 = TOTG * BG
NPA = 10112
RPA = NPA // NS

_mesh = plsc.VectorSubcoreMesh(
    core_axis_name="c", subcore_axis_name="s", num_cores=NC, num_subcores=NS)


@functools.partial(
    pl.kernel,
    out_type=jax.ShapeDtypeStruct((NC, NP), jnp.float32),
    mesh=_mesh,
    scratch_types=[
        pltpu.VMEM((CH, B), jnp.int32),
        pltpu.VMEM((B,), jnp.float32),
        pltpu.VMEM_SHARED((NP,), jnp.float32),
    ],
)
def _deg_kernel(dst_hbm, zeros1_hbm, ones_hbm, deg_out, dst_v, ones_v, deg_sp):
    c = lax.axis_index("c")
    s = lax.axis_index("s")
    wid = c * NS + s
    pltpu.sync_copy(dst_hbm.at[wid], dst_v)
    pltpu.sync_copy(ones_hbm, ones_v)
    pltpu.sync_copy(zeros1_hbm.at[pl.ds(s * RPT, RPT)],
                    deg_sp.at[pl.ds(s * RPT, RPT)])
    plsc.subcore_barrier()

    def body(j, carry):
        pltpu.sync_copy(ones_v, deg_sp.at[dst_v.at[j]], add=True)
        return carry

    lax.fori_loop(0, CH, body, 0)
    plsc.subcore_barrier()
    pltpu.sync_copy(deg_sp.at[pl.ds(s * RPT, RPT)],
                    deg_out.at[c, pl.ds(s * RPT, RPT)])


@functools.partial(
    pl.kernel,
    out_type=jax.ShapeDtypeStruct((NC, NP, D), jnp.float32),
    mesh=_mesh,
    scratch_types=[
        pltpu.VMEM((4, 1, B), jnp.int32),
        pltpu.VMEM((CHMAX, B), jnp.int32),
        pltpu.VMEM((2, B, D), jnp.float32),
        pltpu.VMEM_SHARED((NPA, D), jnp.float32),
        pltpu.SemaphoreType.DMA,
        pltpu.SemaphoreType.DMA,
    ],
)
def _agg_kernel(g_hbm, srcb_hbm, dstb_hbm, zeros2_hbm, agg_out,
                src_r, dst_v, rows, agg_sp, gsem, isem):
    c = lax.axis_index("c")
    s = lax.axis_index("s")
    myc = jnp.where(c == 0, CH0, CH1)
    bbase = pl.multiple_of(
        jnp.where(c == 0, s * CH0, NS * CH0 + s * CH1), 8)
    pltpu.sync_copy(dstb_hbm.at[pl.ds(bbase, CHMAX)], dst_v)
    pltpu.sync_copy(zeros2_hbm.at[pl.ds(s * RPA, RPA)],
                    agg_sp.at[pl.ds(s * RPA, RPA)])
    plsc.subcore_barrier()

    def load_idx(j):
        pltpu.async_copy(srcb_hbm.at[bbase + j], src_r.at[j % 4], isem)

    def wait_idx(j):
        pltpu.make_async_copy(srcb_hbm.at[bbase + j], src_r.at[j % 4],
                              isem).wait()

    def gather(j):
        pltpu.async_copy(g_hbm.at[src_r.at[j % 4, 0]], rows.at[j % 2], gsem)

    load_idx(0)
    load_idx(1)
    wait_idx(0)
    gather(0)

    def body(j, carry):
        @pl.when(j + 1 < myc)
        def _():
            wait_idx(j + 1)
            gather(j + 1)

        @pl.when(j + 2 < myc)
        def _():
            load_idx(j + 2)

        pltpu.make_async_copy(g_hbm.at[src_r.at[j % 4, 0]], rows.at[j % 2],
                              gsem).wait()
        pltpu.sync_copy(rows.at[j % 2], agg_sp.at[dst_v.at[j]], add=True)
        return carry

    lax.fori_loop(0, myc, body, 0)
    plsc.subcore_barrier()
    pltpu.sync_copy(agg_sp.at[pl.ds(s * RPA, RPA)],
                    agg_out.at[c, pl.ds(s * RPA, RPA)])


BM = 1024
_GRID = NP // BM


def _mm_body(x_ref, w_ref, dr_ref, g_ref, dinv_ref):
    deg = dr_ref[0] + dr_ref[1] + 1.0
    dinv = lax.rsqrt(deg)
    h = jnp.dot(x_ref[...], w_ref[...], preferred_element_type=jnp.float32)
    g_ref[...] = dinv * h
    dinv_ref[...] = dinv


_mm = pl.pallas_call(
    _mm_body,
    grid=(_GRID,),
    in_specs=[
        pl.BlockSpec((BM, D), lambda i: (i, 0)),
        pl.BlockSpec((D, D), lambda i: (0, 0)),
        pl.BlockSpec((NC, BM, 1), lambda i: (0, i, 0)),
    ],
    out_specs=[
        pl.BlockSpec((BM, D), lambda i: (i, 0)),
        pl.BlockSpec((BM, 1), lambda i: (i, 0)),
    ],
    out_shape=[
        jax.ShapeDtypeStruct((NP, D), jnp.float32),
        jax.ShapeDtypeStruct((NP, 1), jnp.float32),
    ],
)


def _fin_body(p_ref, g_ref, dinv_ref, b_ref, o_ref):
    ssum = p_ref[0] + p_ref[1] + g_ref[...]
    o_ref[...] = jnp.maximum(dinv_ref[...] * ssum + b_ref[...], 0.0)


_fin = pl.pallas_call(
    _fin_body,
    grid=(_GRID,),
    in_specs=[
        pl.BlockSpec((NC, BM, D), lambda i: (0, i, 0)),
        pl.BlockSpec((BM, D), lambda i: (i, 0)),
        pl.BlockSpec((BM, 1), lambda i: (i, 0)),
        pl.BlockSpec((1, D), lambda i: (0, 0)),
    ],
    out_specs=pl.BlockSpec((BM, D), lambda i: (i, 0)),
    out_shape=jax.ShapeDtypeStruct((NP, D), jnp.float32),
)


def kernel(x, edge_index, W, b):
    src = edge_index[0]
    dst = edge_index[1]
    pad = EPAD2 - E
    fill = jnp.full((pad,), N, jnp.int32)
    srcp = jnp.concatenate([src, fill]).reshape(TOTB, 1, B)
    dstp = jnp.concatenate([dst, fill])
    dstp_deg = dstp[:EPAD].reshape(NW, CH, B)
    dstp_agg = dstp.reshape(TOTB, B)
    xp = jnp.pad(x, ((0, NP - N), (0, 0)))
    zeros1 = jnp.zeros((NP,), jnp.float32)
    zeros2 = jnp.zeros((NP, D), jnp.float32)
    ones = jnp.ones((B,), jnp.float32)

    deg = _deg_kernel(dstp_deg, zeros1, ones)
    g, dinv = _mm(xp, W, deg.reshape(NC, NP, 1))
    agg = _agg_kernel(g, srcp, dstp_agg, zeros2)
    out = _fin(agg, g, dinv, b.reshape(1, D))
    return out[:N]
